# baseline (speedup 1.0000x reference)
import functools

import jax
import jax.numpy as jnp
from jax import lax
from jax.experimental import pallas as pl
from jax.experimental.pallas import tpu as pltpu
from jax.experimental.pallas import tpu_sc as plsc

_N_CODES = 8192
_DIM = 32
_TILE = 2048
_CHUNK = 256
_N_TILES = _N_CODES // _TILE
_BIG_I = 2**30
_BIG_F = 3.0e9
_INF = float("inf")

_NC, _NS = 2, 16
_NW = _NC * _NS
_BPW = 1024 // _NW


def _score_body(x_ref, c_ref, i1_ref, i2_ref, loss_ref, r1, ri1, r2, ri2):
    t = pl.program_id(0)

    @pl.when(t == 0)
    def _init():
        r1[...] = jnp.full((1024, 1), _INF, jnp.float32)
        ri1[...] = jnp.full((1024, 1), _BIG_I, jnp.int32)
        r2[...] = jnp.full((1024, 1), _INF, jnp.float32)
        ri2[...] = jnp.full((1024, 1), _BIG_I, jnp.int32)

    x = x_ref[...]
    ones = jnp.ones((8, _DIM), jnp.float32)

    def half_top2(ct, base):
        dot = lax.dot_general(
            x, ct, (((1,), (1,)), ((), ())),
            preferred_element_type=jnp.float32,
            precision=lax.Precision.HIGHEST,
        )
        cn = lax.dot_general(
            ones, ct * ct, (((1,), (1,)), ((), ())),
            preferred_element_type=jnp.float32,
            precision=lax.Precision.HIGHEST,
        )
        s = cn[0:1, :] - 2.0 * dot
        iota = lax.broadcasted_iota(jnp.int32, (1024, _CHUNK), 1)
        m1 = jnp.min(s, axis=1, keepdims=True)
        i1 = jnp.min(jnp.where(s == m1, iota, _BIG_I), axis=1, keepdims=True)
        s2 = jnp.where(iota == i1, _INF, s)
        m2 = jnp.min(s2, axis=1, keepdims=True)
        i2 = jnp.min(jnp.where(s2 == m2, iota, _BIG_I), axis=1, keepdims=True)
        return m1, i1 + base, m2, i2 + base

    def merge(a, b):
        a1, ia1, a2, ia2 = a
        b1, ib1, b2, ib2 = b
        b_best = b1 < a1
        r1m = jnp.where(b_best, b1, a1)
        ri1m = jnp.where(b_best, ib1, ia1)
        r2m = jnp.where(b_best, jnp.minimum(a1, b2), jnp.minimum(a2, b1))
        ri2m = jnp.where(
            b_best,
            jnp.where(b2 < a1, ib2, ia1),
            jnp.where(b1 < a2, ib1, ia2),
        )
        return r1m, ri1m, r2m, ri2m

    chunks = [
        half_top2(c_ref[k * _CHUNK:(k + 1) * _CHUNK, :],
                  t * _TILE + k * _CHUNK)
        for k in range(_TILE // _CHUNK)
    ]
    while len(chunks) > 1:
        chunks = [merge(chunks[i], chunks[i + 1])
                  for i in range(0, len(chunks), 2)]
    nr1, nri1, nr2, nri2 = merge(
        (r1[...], ri1[...], r2[...], ri2[...]), chunks[0])
    r1[...], ri1[...], r2[...], ri2[...] = nr1, nri1, nr2, nri2

    @pl.when(t == _N_TILES - 1)
    def _finalize():
        i1_ref[...] = ri1[...]
        i2_ref[...] = ri2[...]
        loss_ref[...] = (
            1.25 * (jnp.sum(r1[...]) + jnp.sum(x * x)) / (1024.0 * _DIM)
        ).reshape(1, 1)


def _score(x, codewords):
    return pl.pallas_call(
        _score_body,
        grid=(_N_TILES,),
        in_specs=[
            pl.BlockSpec((1024, _DIM), lambda t: (0, 0)),
            pl.BlockSpec((_TILE, _DIM), lambda t: (t, 0)),
        ],
        out_specs=[
            pl.BlockSpec((1024, 1), lambda t: (0, 0)),
            pl.BlockSpec((1024, 1), lambda t: (0, 0)),
            pl.BlockSpec((1, 1), lambda t: (0, 0)),
        ],
        out_shape=(
            jax.ShapeDtypeStruct((1024, 1), jnp.int32),
            jax.ShapeDtypeStruct((1024, 1), jnp.int32),
            jax.ShapeDtypeStruct((1, 1), jnp.float32),
        ),
        scratch_shapes=[
            pltpu.VMEM((1024, 1), jnp.float32),
            pltpu.VMEM((1024, 1), jnp.int32),
            pltpu.VMEM((1024, 1), jnp.float32),
            pltpu.VMEM((1024, 1), jnp.int32),
        ],
    )(x, codewords)


def _rerank_body(c_hbm, i1_hbm, i2_hbm, x_hbm, q_hbm, idx_hbm,
                 i1_v, i2_v, iq1_v, iq2_v, x_v, c1_v, c2_v, q_v, idx_v,
                 sem1, sem2):
    wid = lax.axis_index("s") * _NC + lax.axis_index("c")
    base = wid * _BPW
    pltpu.sync_copy(i1_hbm.at[pl.ds(base, _BPW)], i1_v)
    pltpu.sync_copy(i2_hbm.at[pl.ds(base, _BPW)], i2_v)
    pltpu.sync_copy(x_hbm.at[pl.ds(base * _DIM, _BPW * _DIM)], x_v)
    for g in range(_BPW // 16):
        iq1_v[pl.ds(g * 16, 16)] = lax.shift_right_logical(
            i1_v[pl.ds(g * 16, 16)], 2)
        iq2_v[pl.ds(g * 16, 16)] = lax.shift_right_logical(
            i2_v[pl.ds(g * 16, 16)], 2)
    cp1 = pltpu.make_async_copy(c_hbm.at[iq1_v], c1_v, sem1)
    cp2 = pltpu.make_async_copy(c_hbm.at[iq2_v], c2_v, sem2)
    cp1.start()
    cp2.start()
    cp1.wait()
    cp2.wait()
    lane = lax.iota(jnp.int32, 16)

    def permute(v, idx):
        return lax.gather(
            v, idx[:, None],
            dimension_numbers=lax.GatherDimensionNumbers(
                offset_dims=(), collapsed_slice_dims=(0,),
                start_index_map=(0,)),
            slice_sizes=(1,),
            mode=lax.GatherScatterMode.PROMISE_IN_BOUNDS,
        )

    def allsum(v):
        for m in (1, 2, 4, 8):
            v = v + permute(v, lane ^ m)
        return v

    for h in range(_BPW // 16):
        iv1 = i1_v[pl.ds(h * 16, 16)]
        iv2 = i2_v[pl.ds(h * 16, 16)]
        cur = iv2
        for j in range(16):
            t = h * 16 + j
            xa = x_v[pl.ds(t * _DIM, 16)]
            xb = x_v[pl.ds(t * _DIM + 16, 16)]
            i1t, i2t = iv1[j], iv2[j]
            off1 = (i1t & 3) * _DIM
            off2 = (i2t & 3) * _DIM
            c1a = c1_v[t, pl.ds(off1, 16)]
            c1b = c1_v[t, pl.ds(off1 + 16, 16)]
            c2a = c2_v[t, pl.ds(off2, 16)]
            c2b = c2_v[t, pl.ds(off2 + 16, 16)]
            e1a, e1b = xa - c1a, xb - c1b
            e2a, e2b = xa - c2a, xb - c2b
            d1 = allsum(e1a * e1a + e1b * e1b)[0]
            d2 = allsum(e2a * e2a + e2b * e2b)[0]
            pick1 = (d1 < d2) | ((d1 == d2) & (i1t < i2t))

            @pl.when(pick1)
            def _take1(xa=xa, xb=xb, c1a=c1a, c1b=c1b, t=t):
                q_v[pl.ds(t * _DIM, 16)] = xa + (c1a - xa)
                q_v[pl.ds(t * _DIM + 16, 16)] = xb + (c1b - xb)

            @pl.when(jnp.logical_not(pick1))
            def _take2(xa=xa, xb=xb, c2a=c2a, c2b=c2b, t=t):
                q_v[pl.ds(t * _DIM, 16)] = xa + (c2a - xa)
                q_v[pl.ds(t * _DIM + 16, 16)] = xb + (c2b - xb)

            onehot = 1 - jnp.minimum(jnp.abs(lane - j), 1)
            selj = jnp.where(pick1, i1t, i2t)
            cur = cur + (selj - i2t) * onehot
        idx_v[pl.ds(h * 16, 16)] = cur
    pltpu.sync_copy(q_v, q_hbm.at[pl.ds(base * _DIM, _BPW * _DIM)])
    pltpu.sync_copy(idx_v, idx_hbm.at[pl.ds(base, _BPW)])


@functools.cache
def _rerank():
    return pl.kernel(
        _rerank_body,
        out_type=(
            jax.ShapeDtypeStruct((1024 * _DIM,), jnp.float32),
            jax.ShapeDtypeStruct((1024,), jnp.int32),
        ),
        mesh=plsc.VectorSubcoreMesh(core_axis_name="c", subcore_axis_name="s",
                                    num_cores=_NC, num_subcores=_NS),
        scratch_types=[
            pltpu.VMEM((_BPW,), jnp.int32),
            pltpu.VMEM((_BPW,), jnp.int32),
            pltpu.VMEM((_BPW,), jnp.int32),
            pltpu.VMEM((_BPW,), jnp.int32),
            pltpu.VMEM((_BPW * _DIM,), jnp.float32),
            pltpu.VMEM((_BPW, 128), jnp.float32),
            pltpu.VMEM((_BPW, 128), jnp.float32),
            pltpu.VMEM((_BPW * _DIM,), jnp.float32),
            pltpu.VMEM((_BPW,), jnp.int32),
            pltpu.SemaphoreType.DMA,
            pltpu.SemaphoreType.DMA,
        ],
    )


@jax.jit
def kernel(inputs, codewords):
    shape = inputs.shape
    x = inputs.reshape(-1, _DIM)
    i1, i2, loss = _score(x, codewords)
    cw4 = codewords.reshape(_N_CODES // 4, 4 * _DIM)
    q, idx = _rerank()(cw4, i1.reshape(-1), i2.reshape(-1),
                       x.reshape(-1))
    return (q.reshape(shape), idx.reshape(shape[:-1]), loss[0, 0])


# speedup vs baseline: 1.0488x; 1.0488x over previous
import functools

import jax
import jax.numpy as jnp
from jax import lax
from jax.experimental import pallas as pl
from jax.experimental.pallas import tpu as pltpu
from jax.experimental.pallas import tpu_sc as plsc

_N_CODES = 8192
_DIM = 32
_TILE = 4096
_CHUNK = 512
_N_TILES = _N_CODES // _TILE
_BIG_I = 2**30
_BIG_F = 3.0e9
_INF = float("inf")

_NC, _NS = 2, 16
_NW = _NC * _NS
_BPW = 1024 // _NW


def _score_body(x_ref, c_ref, i1_ref, i2_ref, loss_ref, r1, ri1, r2, ri2):
    t = pl.program_id(0)

    @pl.when(t == 0)
    def _init():
        r1[...] = jnp.full((1024, 1), _INF, jnp.float32)
        ri1[...] = jnp.full((1024, 1), _BIG_I, jnp.int32)
        r2[...] = jnp.full((1024, 1), _INF, jnp.float32)
        ri2[...] = jnp.full((1024, 1), _BIG_I, jnp.int32)

    x = x_ref[...]
    ones = jnp.ones((8, _DIM), jnp.float32)

    def half_top2(ct, base):
        dot = lax.dot_general(
            x, ct, (((1,), (1,)), ((), ())),
            preferred_element_type=jnp.float32,
            precision=lax.Precision.HIGHEST,
        )
        cn = lax.dot_general(
            ones, ct * ct, (((1,), (1,)), ((), ())),
            preferred_element_type=jnp.float32,
            precision=lax.Precision.HIGHEST,
        )
        s = cn[0:1, :] - 2.0 * dot
        iota = lax.broadcasted_iota(jnp.int32, (1024, _CHUNK), 1)
        m1 = jnp.min(s, axis=1, keepdims=True)
        i1 = jnp.min(jnp.where(s == m1, iota, _BIG_I), axis=1, keepdims=True)
        s2 = jnp.where(iota == i1, _INF, s)
        m2 = jnp.min(s2, axis=1, keepdims=True)
        i2 = jnp.min(jnp.where(s2 == m2, iota, _BIG_I), axis=1, keepdims=True)
        return m1, i1 + base, m2, i2 + base

    def merge(a, b):
        a1, ia1, a2, ia2 = a
        b1, ib1, b2, ib2 = b
        b_best = b1 < a1
        r1m = jnp.where(b_best, b1, a1)
        ri1m = jnp.where(b_best, ib1, ia1)
        r2m = jnp.where(b_best, jnp.minimum(a1, b2), jnp.minimum(a2, b1))
        ri2m = jnp.where(
            b_best,
            jnp.where(b2 < a1, ib2, ia1),
            jnp.where(b1 < a2, ib1, ia2),
        )
        return r1m, ri1m, r2m, ri2m

    chunks = [
        half_top2(c_ref[k * _CHUNK:(k + 1) * _CHUNK, :],
                  t * _TILE + k * _CHUNK)
        for k in range(_TILE // _CHUNK)
    ]
    while len(chunks) > 1:
        chunks = [merge(chunks[i], chunks[i + 1])
                  for i in range(0, len(chunks), 2)]
    nr1, nri1, nr2, nri2 = merge(
        (r1[...], ri1[...], r2[...], ri2[...]), chunks[0])
    r1[...], ri1[...], r2[...], ri2[...] = nr1, nri1, nr2, nri2

    @pl.when(t == _N_TILES - 1)
    def _finalize():
        i1_ref[...] = ri1[...]
        i2_ref[...] = ri2[...]
        loss_ref[...] = (
            1.25 * (jnp.sum(r1[...]) + jnp.sum(x * x)) / (1024.0 * _DIM)
        ).reshape(1, 1)


def _score(x, codewords):
    return pl.pallas_call(
        _score_body,
        grid=(_N_TILES,),
        in_specs=[
            pl.BlockSpec((1024, _DIM), lambda t: (0, 0)),
            pl.BlockSpec((_TILE, _DIM), lambda t: (t, 0)),
        ],
        out_specs=[
            pl.BlockSpec((1024, 1), lambda t: (0, 0)),
            pl.BlockSpec((1024, 1), lambda t: (0, 0)),
            pl.BlockSpec((1, 1), lambda t: (0, 0)),
        ],
        out_shape=(
            jax.ShapeDtypeStruct((1024, 1), jnp.int32),
            jax.ShapeDtypeStruct((1024, 1), jnp.int32),
            jax.ShapeDtypeStruct((1, 1), jnp.float32),
        ),
        scratch_shapes=[
            pltpu.VMEM((1024, 1), jnp.float32),
            pltpu.VMEM((1024, 1), jnp.int32),
            pltpu.VMEM((1024, 1), jnp.float32),
            pltpu.VMEM((1024, 1), jnp.int32),
        ],
    )(x, codewords)


def _rerank_body(c_hbm, i1_hbm, i2_hbm, x_hbm, q_hbm, idx_hbm,
                 i1_v, i2_v, iq1_v, iq2_v, x_v, c1_v, c2_v, q_v, idx_v,
                 sem1, sem2):
    wid = lax.axis_index("s") * _NC + lax.axis_index("c")
    base = wid * _BPW
    pltpu.sync_copy(i1_hbm.at[pl.ds(base, _BPW)], i1_v)
    pltpu.sync_copy(i2_hbm.at[pl.ds(base, _BPW)], i2_v)
    pltpu.sync_copy(x_hbm.at[pl.ds(base * _DIM, _BPW * _DIM)], x_v)
    for g in range(_BPW // 16):
        iq1_v[pl.ds(g * 16, 16)] = lax.shift_right_logical(
            i1_v[pl.ds(g * 16, 16)], 2)
        iq2_v[pl.ds(g * 16, 16)] = lax.shift_right_logical(
            i2_v[pl.ds(g * 16, 16)], 2)
    cp1 = pltpu.make_async_copy(c_hbm.at[iq1_v], c1_v, sem1)
    cp2 = pltpu.make_async_copy(c_hbm.at[iq2_v], c2_v, sem2)
    cp1.start()
    cp2.start()
    cp1.wait()
    cp2.wait()
    lane = lax.iota(jnp.int32, 16)

    def permute(v, idx):
        return lax.gather(
            v, idx[:, None],
            dimension_numbers=lax.GatherDimensionNumbers(
                offset_dims=(), collapsed_slice_dims=(0,),
                start_index_map=(0,)),
            slice_sizes=(1,),
            mode=lax.GatherScatterMode.PROMISE_IN_BOUNDS,
        )

    def allsum(v):
        for m in (1, 2, 4, 8):
            v = v + permute(v, lane ^ m)
        return v

    for h in range(_BPW // 16):
        iv1 = i1_v[pl.ds(h * 16, 16)]
        iv2 = i2_v[pl.ds(h * 16, 16)]
        cur = iv2
        for j in range(16):
            t = h * 16 + j
            xa = x_v[pl.ds(t * _DIM, 16)]
            xb = x_v[pl.ds(t * _DIM + 16, 16)]
            i1t, i2t = iv1[j], iv2[j]
            off1 = (i1t & 3) * _DIM
            off2 = (i2t & 3) * _DIM
            c1a = c1_v[t, pl.ds(off1, 16)]
            c1b = c1_v[t, pl.ds(off1 + 16, 16)]
            c2a = c2_v[t, pl.ds(off2, 16)]
            c2b = c2_v[t, pl.ds(off2 + 16, 16)]
            e1a, e1b = xa - c1a, xb - c1b
            e2a, e2b = xa - c2a, xb - c2b
            d1 = allsum(e1a * e1a + e1b * e1b)[0]
            d2 = allsum(e2a * e2a + e2b * e2b)[0]
            pick1 = (d1 < d2) | ((d1 == d2) & (i1t < i2t))

            @pl.when(pick1)
            def _take1(xa=xa, xb=xb, c1a=c1a, c1b=c1b, t=t):
                q_v[pl.ds(t * _DIM, 16)] = xa + (c1a - xa)
                q_v[pl.ds(t * _DIM + 16, 16)] = xb + (c1b - xb)

            @pl.when(jnp.logical_not(pick1))
            def _take2(xa=xa, xb=xb, c2a=c2a, c2b=c2b, t=t):
                q_v[pl.ds(t * _DIM, 16)] = xa + (c2a - xa)
                q_v[pl.ds(t * _DIM + 16, 16)] = xb + (c2b - xb)

            onehot = 1 - jnp.minimum(jnp.abs(lane - j), 1)
            selj = jnp.where(pick1, i1t, i2t)
            cur = cur + (selj - i2t) * onehot
        idx_v[pl.ds(h * 16, 16)] = cur
    pltpu.sync_copy(q_v, q_hbm.at[pl.ds(base * _DIM, _BPW * _DIM)])
    pltpu.sync_copy(idx_v, idx_hbm.at[pl.ds(base, _BPW)])


@functools.cache
def _rerank():
    return pl.kernel(
        _rerank_body,
        out_type=(
            jax.ShapeDtypeStruct((1024 * _DIM,), jnp.float32),
            jax.ShapeDtypeStruct((1024,), jnp.int32),
        ),
        mesh=plsc.VectorSubcoreMesh(core_axis_name="c", subcore_axis_name="s",
                                    num_cores=_NC, num_subcores=_NS),
        scratch_types=[
            pltpu.VMEM((_BPW,), jnp.int32),
            pltpu.VMEM((_BPW,), jnp.int32),
            pltpu.VMEM((_BPW,), jnp.int32),
            pltpu.VMEM((_BPW,), jnp.int32),
            pltpu.VMEM((_BPW * _DIM,), jnp.float32),
            pltpu.VMEM((_BPW, 128), jnp.float32),
            pltpu.VMEM((_BPW, 128), jnp.float32),
            pltpu.VMEM((_BPW * _DIM,), jnp.float32),
            pltpu.VMEM((_BPW,), jnp.int32),
            pltpu.SemaphoreType.DMA,
            pltpu.SemaphoreType.DMA,
        ],
    )


@jax.jit
def kernel(inputs, codewords):
    shape = inputs.shape
    x = inputs.reshape(-1, _DIM)
    i1, i2, loss = _score(x, codewords)
    cw4 = codewords.reshape(_N_CODES // 4, 4 * _DIM)
    q, idx = _rerank()(cw4, i1.reshape(-1), i2.reshape(-1),
                       x.reshape(-1))
    return (q.reshape(shape), idx.reshape(shape[:-1]), loss[0, 0])
